# baseline (speedup 1.0000x reference)
import jax
import jax.numpy as jnp
from jax import lax
from jax.experimental import pallas as pl

_BB = 4096


def kernel(input_ids, unigram):
    batch, hist = input_ids.shape
    vocab, dim = unigram.shape
    lhs = unigram.T.astype(jnp.bfloat16)
    ids3 = input_ids.T.reshape(hist, 1, batch)

    def body(lhs_ref, ids_ref, out_ref):
        idb = ids_ref[0, 0, :].astype(jnp.int16)
        oh = lax.broadcasted_iota(jnp.int16, (vocab, _BB), 0) == idb[None, :]
        out_ref[0] = jnp.dot(
            lhs_ref[...], oh.astype(jnp.bfloat16),
            preferred_element_type=jnp.float32,
        )

    out_t = pl.pallas_call(
        body,
        grid=(hist, batch // _BB),
        in_specs=[
            pl.BlockSpec((dim, vocab), lambda h, b: (0, 0)),
            pl.BlockSpec((1, 1, _BB), lambda h, b: (h, 0, b)),
        ],
        out_specs=pl.BlockSpec((1, dim, _BB), lambda h, b: (h, 0, b)),
        out_shape=jax.ShapeDtypeStruct((hist, dim, batch), jnp.float32),
    )(lhs, ids3)
    return out_t.transpose(2, 0, 1)


# speedup vs baseline: 1.0066x; 1.0066x over previous
import jax
import jax.numpy as jnp
from jax import lax
from jax.experimental import pallas as pl

_BB = 4096


def kernel(input_ids, unigram):
    batch, hist = input_ids.shape
    vocab, dim = unigram.shape
    lhs = unigram.T.astype(jnp.bfloat16)
    ids3 = input_ids.T.reshape(hist, 1, batch)

    def body(lhs_ref, ids_ref, out_ref):
        idb = ids_ref[0, 0, :]
        oh = lax.broadcasted_iota(jnp.int32, (vocab, _BB), 0) == idb[None, :]
        out_ref[0] = jnp.dot(
            lhs_ref[...], oh.astype(jnp.bfloat16),
            preferred_element_type=jnp.float32,
        )

    out_t = pl.pallas_call(
        body,
        grid=(hist, batch // _BB),
        in_specs=[
            pl.BlockSpec((dim, vocab), lambda h, b: (0, 0)),
            pl.BlockSpec((1, 1, _BB), lambda h, b: (h, 0, b)),
        ],
        out_specs=pl.BlockSpec((1, dim, _BB), lambda h, b: (h, 0, b)),
        out_shape=jax.ShapeDtypeStruct((hist, dim, batch), jnp.float32),
    )(lhs, ids3)
    return out_t.transpose(2, 0, 1)
